# baseline (speedup 1.0000x reference)
import functools

import jax
import jax.numpy as jnp
from jax import lax
from jax.experimental import pallas as pl
from jax.experimental.pallas import tpu as pltpu
from jax.experimental.pallas import tpu_sc as plsc

VOCAB = 100000
EMBED = 128
BATCH = 4096
CTX = 20

NC = 2
NS = 16
NW = NC * NS
RPW = BATCH // NW
LANES = 16

def _pool_body(idxT_hbm, tab_hbm, x_hbm, idx_v, rows0_v, rows1_v, acc_v,
               sem_a, sem0, sem1):
    wid = lax.axis_index("s") * NC + lax.axis_index("c")
    base = wid * RPW
    rows = (rows0_v, rows1_v)
    sems = (sem0, sem1)
    pltpu.sync_copy(idxT_hbm.at[:, pl.ds(base, RPW)], idx_v)
    cp_acc = pltpu.async_copy(tab_hbm.at[idx_v.at[0]], acc_v, sem_a)
    pending = [None, None]
    pending[1] = pltpu.async_copy(tab_hbm.at[idx_v.at[1]], rows[1], sems[1])
    cp_acc.wait()

    def accum(buf):
        def body(r, _):
            for u in range(2):
                for c in range(EMBED // LANES):
                    sl = pl.ds(c * LANES, LANES)
                    plsc.addupdate(acc_v.at[2 * r + u, sl], buf[2 * r + u, sl])
            return 0

        lax.fori_loop(0, RPW // 2, body, 0)

    for j in range(1, CTX):
        cur = j % 2
        pending[cur].wait()
        if j + 1 < CTX:
            nxt = (j + 1) % 2
            pending[nxt] = pltpu.async_copy(
                tab_hbm.at[idx_v.at[j + 1]], rows[nxt], sems[nxt]
            )
        accum(rows[cur])

    pltpu.sync_copy(acc_v, x_hbm.at[pl.ds(base, RPW), :])


@functools.cache
def _pool_kernel():
    mesh = plsc.VectorSubcoreMesh(
        core_axis_name="c", subcore_axis_name="s", num_cores=NC, num_subcores=NS
    )
    return pl.kernel(
        _pool_body,
        out_type=jax.ShapeDtypeStruct((BATCH, EMBED), jnp.float32),
        mesh=mesh,
        scratch_types=[
            pltpu.VMEM((CTX, RPW), jnp.int32),
            pltpu.VMEM((RPW, EMBED), jnp.float32),
            pltpu.VMEM((RPW, EMBED), jnp.float32),
            pltpu.VMEM((RPW, EMBED), jnp.float32),
            pltpu.SemaphoreType.DMA,
            pltpu.SemaphoreType.DMA,
            pltpu.SemaphoreType.DMA,
        ],
    )


BN = 1024
NBLK = (VOCAB + BN - 1) // BN


def _mm_body(x_ref, w_ref, b_ref, o_ref):
    acc = lax.dot_general(
        w_ref[...], x_ref[...], (((1,), (1,)), ((), ())),
        preferred_element_type=jnp.float32,
    )
    o_ref[...] = acc * (1.0 / CTX) + b_ref[...].T


def _matmul(x, fc_w, fc_b2d):
    return pl.pallas_call(
        _mm_body,
        grid=(NBLK,),
        in_specs=[
            pl.BlockSpec((BATCH, EMBED), lambda i: (0, 0)),
            pl.BlockSpec((BN, EMBED), lambda i: (i, 0)),
            pl.BlockSpec((1, BN), lambda i: (0, i)),
        ],
        out_specs=pl.BlockSpec((BN, BATCH), lambda i: (i, 0)),
        out_shape=jax.ShapeDtypeStruct((VOCAB, BATCH), jnp.float32),
        compiler_params=pltpu.CompilerParams(
            dimension_semantics=("arbitrary",),
        ),
    )(x, fc_w, fc_b2d)


def kernel(inputs, emb_table, fc_w, fc_b):
    idxT = inputs.T.astype(jnp.int32)
    w16 = fc_w.astype(jnp.bfloat16)
    x = _pool_kernel()(idxT, emb_table)
    logits_t = _matmul(x.astype(jnp.bfloat16), w16, fc_b.reshape(1, VOCAB))
    return logits_t.T


# speedup vs baseline: 1.0201x; 1.0201x over previous
import functools

import jax
import jax.numpy as jnp
from jax import lax
from jax.experimental import pallas as pl
from jax.experimental.pallas import tpu as pltpu
from jax.experimental.pallas import tpu_sc as plsc

VOCAB = 100000
EMBED = 128
BATCH = 4096
CTX = 20

NC = 2
NS = 16
NW = NC * NS
RPW = BATCH // NW
LANES = 16

NBUF = 4


def _pool_body(idxT_hbm, tab_hbm, x_hbm, idx_v, r0, r1, r2, r3, acc_v,
               sem_a, s0, s1, s2, s3):
    wid = lax.axis_index("s") * NC + lax.axis_index("c")
    base = wid * RPW
    rows = (r0, r1, r2, r3)
    sems = (s0, s1, s2, s3)
    pltpu.sync_copy(idxT_hbm.at[:, pl.ds(base, RPW)], idx_v)
    cp_acc = pltpu.async_copy(tab_hbm.at[idx_v.at[0]], acc_v, sem_a)
    pending = [None] * NBUF
    for j in range(1, 1 + NBUF):
        b = (j - 1) % NBUF
        pending[b] = pltpu.async_copy(tab_hbm.at[idx_v.at[j]], rows[b], sems[b])
    cp_acc.wait()

    def accum(buf):
        def body(r, _):
            for u in range(2):
                for c in range(EMBED // LANES):
                    sl = pl.ds(c * LANES, LANES)
                    plsc.addupdate(acc_v.at[2 * r + u, sl], buf[2 * r + u, sl])
            return 0

        lax.fori_loop(0, RPW // 2, body, 0)

    for j in range(1, CTX):
        b = (j - 1) % NBUF
        pending[b].wait()
        accum(rows[b])
        nj = j + NBUF
        if nj < CTX:
            pending[b] = pltpu.async_copy(
                tab_hbm.at[idx_v.at[nj]], rows[b], sems[b]
            )

    pltpu.sync_copy(acc_v, x_hbm.at[pl.ds(base, RPW), :])


@functools.cache
def _pool_kernel():
    mesh = plsc.VectorSubcoreMesh(
        core_axis_name="c", subcore_axis_name="s", num_cores=NC, num_subcores=NS
    )
    return pl.kernel(
        _pool_body,
        out_type=jax.ShapeDtypeStruct((BATCH, EMBED), jnp.float32),
        mesh=mesh,
        scratch_types=[
            pltpu.VMEM((CTX, RPW), jnp.int32),
            pltpu.VMEM((RPW, EMBED), jnp.float32),
            pltpu.VMEM((RPW, EMBED), jnp.float32),
            pltpu.VMEM((RPW, EMBED), jnp.float32),
            pltpu.VMEM((RPW, EMBED), jnp.float32),
            pltpu.VMEM((RPW, EMBED), jnp.float32),
            pltpu.SemaphoreType.DMA,
            pltpu.SemaphoreType.DMA,
            pltpu.SemaphoreType.DMA,
            pltpu.SemaphoreType.DMA,
            pltpu.SemaphoreType.DMA,
        ],
    )


BN = 1024
NBLK = (VOCAB + BN - 1) // BN


def _mm_body(x_ref, w_ref, b_ref, o_ref):
    acc = lax.dot_general(
        w_ref[...], x_ref[...], (((1,), (1,)), ((), ())),
        preferred_element_type=jnp.float32,
    )
    o_ref[...] = acc * (1.0 / CTX) + b_ref[...].T


def _matmul(x, fc_w, fc_b2d):
    return pl.pallas_call(
        _mm_body,
        grid=(NBLK,),
        in_specs=[
            pl.BlockSpec((BATCH, EMBED), lambda i: (0, 0)),
            pl.BlockSpec((BN, EMBED), lambda i: (i, 0)),
            pl.BlockSpec((1, BN), lambda i: (0, i)),
        ],
        out_specs=pl.BlockSpec((BN, BATCH), lambda i: (i, 0)),
        out_shape=jax.ShapeDtypeStruct((VOCAB, BATCH), jnp.float32),
        compiler_params=pltpu.CompilerParams(
            dimension_semantics=("arbitrary",),
        ),
    )(x, fc_w, fc_b2d)


def kernel(inputs, emb_table, fc_w, fc_b):
    idxT = inputs.T.astype(jnp.int32)
    x = _pool_kernel()(idxT, emb_table)
    logits_t = _matmul(x, fc_w, fc_b.reshape(1, VOCAB))
    return logits_t.T
